# baseline (speedup 1.0000x reference)
import jax
import jax.numpy as jnp
from jax.experimental import pallas as pl
import sc_impl


def kernel(x, W):
    return sc_impl.sc_kernel(x, W)


# speedup vs baseline: 10.3529x; 10.3529x over previous
import jax
import jax.numpy as jnp
from jax.experimental import pallas as pl
from jax.experimental.pallas import tpu as pltpu


_BLK = 8192


def _mm_body(w_ref, xt_ref, o_ref):
    o_ref[...] = jnp.dot(w_ref[...], xt_ref[...],
                         preferred_element_type=jnp.float32)


def kernel(x, W):
    B, I = x.shape
    J = W.shape[0]
    xt = x.T
    outT = pl.pallas_call(
        _mm_body,
        grid=(B // _BLK,),
        in_specs=[
            pl.BlockSpec((J, I), lambda g: (0, 0)),
            pl.BlockSpec((I, _BLK), lambda g: (0, g)),
        ],
        out_specs=pl.BlockSpec((J, _BLK), lambda g: (0, g)),
        out_shape=jax.ShapeDtypeStruct((J, B), jnp.float32),
        compiler_params=pltpu.CompilerParams(
            dimension_semantics=("parallel",),
        ),
    )(W, xt)
    return outT.T
